# baseline (speedup 1.0000x reference)
import functools

import jax
import jax.numpy as jnp
from jax import lax
from jax.experimental import pallas as pl
from jax.experimental.pallas import tpu as pltpu
from jax.experimental.pallas import tpu_sc as plsc

BATCH = 16384
FIELDS = 26
D = 64
NC, NS = 2, 16
NW = NC * NS
CHUNK = 128
NUNITS = FIELDS * (BATCH // CHUNK)
UPW = NUNITS // NW
NFG = D // 16
OUT_ELEMS = BATCH * FIELDS * D

_mesh = plsc.VectorSubcoreMesh(core_axis_name="c", subcore_axis_name="s")

_scratch = (
    [pltpu.VMEM((UPW, CHUNK), jnp.int32)]
    + [pltpu.VMEM((CHUNK, D // 2), jnp.float32)] * 4
    + [pltpu.VMEM((16, CHUNK + 1), jnp.float32)] * (2 * NFG)
    + [pltpu.SemaphoreType.DMA] * 2
    + [pltpu.SemaphoreType.DMA] * 2
)


@functools.partial(
    pl.kernel,
    mesh=_mesh,
    out_type=jax.ShapeDtypeStruct((OUT_ELEMS // CHUNK, CHUNK), jnp.float32),
    scratch_types=_scratch,
    compiler_params=pltpu.CompilerParams(
        needs_layout_passes=False, use_tc_tiling_on_sc=False),
)
def _gather_k(tlo_hbm, thi_hbm, x_hbm, out_hbm, idx_v, *rest):
    gbufs = ((rest[0], rest[2]), (rest[1], rest[3]))
    tbufs = (rest[4:4 + NFG], rest[4 + NFG:4 + 2 * NFG])
    gsems = rest[4 + 2 * NFG:6 + 2 * NFG]
    wsems = rest[6 + 2 * NFG:8 + 2 * NFG]
    halves = (tlo_hbm, thi_hbm)

    w = lax.axis_index("s") * NC + lax.axis_index("c")
    ubase = w * UPW
    pltpu.sync_copy(x_hbm.at[pl.ds(ubase, UPW)], idx_v)

    rows16 = lax.iota(jnp.int32, 16)
    zeros16 = rows16 * 0

    def wait_gather(s):
        for h in range(2):
            pltpu.make_async_copy(
                halves[h].at[idx_v.at[0]], gbufs[s][h], gsems[s]).wait()

    def wait_writes(s):
        for fg in range(NFG):
            for _ in range(2):
                pltpu.make_async_copy(
                    tbufs[s][fg].at[pl.ds(0, 8), pl.ds(0, CHUNK)],
                    out_hbm.at[pl.ds(0, 8)], wsems[s]).wait()

    def transpose_unit(s):
        @plsc.parallel_loop(0, CHUNK, unroll=8)
        def b_body(b):
            cols = zeros16 + b
            for fg in range(NFG):
                vals = gbufs[s][fg // 2][b, pl.ds((fg % 2) * 16, 16)]
                plsc.store_scatter(tbufs[s][fg], [rows16, cols], vals)

    def write_unit(s, u):
        j = u // (BATCH // CHUNK)
        c = u % (BATCH // CHUNK)
        rbase = j * (64 * 128) + c * 8
        for r in range(8):
            pltpu.async_copy(
                tbufs[s][r // 2].at[pl.ds(8 * r % 16, 8), pl.ds(0, CHUNK)],
                out_hbm.at[pl.ds(rbase + r * (128 * 8), 8)],
                wsems[s])

    def issue_gather(s, u):
        for h in range(2):
            pltpu.async_copy(halves[h].at[idx_v.at[u]], gbufs[s][h], gsems[s])

    issue_gather(0, 0)
    issue_gather(1, 1)
    for s in range(2):
        wait_gather(s)
        transpose_unit(s)
        write_unit(s, ubase + s)
        issue_gather(s, s + 2)

    def lap(L, carry):
        for s in range(2):
            u = 2 * L + s
            wait_writes(s)
            wait_gather(s)
            transpose_unit(s)
            write_unit(s, ubase + u)
            issue_gather(s, u + 2)
        return carry

    lax.fori_loop(1, UPW // 2 - 1, lap, 0)

    for s in range(2):
        u = UPW - 2 + s
        wait_writes(s)
        wait_gather(s)
        transpose_unit(s)
        write_unit(s, ubase + u)

    for s in range(2):
        wait_writes(s)


def kernel(x, table):
    idx = x.astype(jnp.int32).T.reshape(NUNITS, CHUNK)
    out = _gather_k(table[:, :D // 2], table[:, D // 2:], idx)
    o = out.reshape(FIELDS, 8, 128, 8, 128)
    o = o.transpose(2, 4, 0, 1, 3)
    return o.reshape(BATCH, FIELDS, D)


# speedup vs baseline: 1.2261x; 1.2261x over previous
import functools

import jax
import jax.numpy as jnp
from jax import lax
from jax.experimental import pallas as pl
from jax.experimental.pallas import tpu as pltpu
from jax.experimental.pallas import tpu_sc as plsc

BATCH = 16384
FIELDS = 26
D = 64
ACT = 1000000
NC, NS = 2, 16
NW = NC * NS
OUT_ELEMS = BATCH * FIELDS * D

CC = 256
MAIN = 999936
NCHK = MAIN // CC
TAIL = ACT - MAIN
NPAIR = ACT // 2
ITER = (NCHK + NW - 1) // NW + 1

CHUNK = 128
NUNITS = FIELDS * (BATCH // CHUNK)
UPW = NUNITS // NW
NFG = D // 16

_mesh = plsc.VectorSubcoreMesh(core_axis_name="c", subcore_axis_name="s")

_r_scratch = (
    [pltpu.VMEM((D, CC + 1), jnp.float32)] * 2
    + [pltpu.VMEM((CC // 2, 128), jnp.float32)] * 2
    + [pltpu.VMEM((D, 129), jnp.float32)]
    + [pltpu.SemaphoreType.DMA] * 2
    + [pltpu.SemaphoreType.DMA] * 2
)


@functools.partial(
    pl.kernel,
    mesh=_mesh,
    out_type=jax.ShapeDtypeStruct((NPAIR, 128), jnp.float32),
    scratch_types=_r_scratch,
    compiler_params=pltpu.CompilerParams(needs_layout_passes=False),
)
def _repack_k(tT_hbm, tail_hbm, out_hbm, in0, in1, tb0, tb1, tailb,
              i0, i1, o0, o1):
    w = lax.axis_index("s") * NC + lax.axis_index("c")
    inbufs = (in0, in1)
    tbufs = (tb0, tb1)
    isems = (i0, i1)
    osems = (o0, o1)

    it = lax.iota(jnp.int32, 16)
    zeros16 = it * 0
    rowpats = [it + 16 * fg for fg in range(D // 16)]

    def issue_in(s, t):
        for r in range(8):
            pltpu.async_copy(
                tT_hbm.at[pl.ds(8 * r, 8), pl.ds(t * CC, CC)],
                inbufs[s].at[pl.ds(8 * r, 8), pl.ds(0, CC)], isems[s])

    def wait_in(s):
        for r in range(8):
            pltpu.make_async_copy(
                tT_hbm.at[pl.ds(0, 8), pl.ds(0, CC)],
                inbufs[s].at[pl.ds(0, 8), pl.ds(0, CC)], isems[s]).wait()

    def transpose_chunk(s):
        @plsc.parallel_loop(0, CC, unroll=4)
        def c_body(c):
            q = c >> 1
            h = (c & 1) * 64
            cols = zeros16 + c
            for fg in range(D // 16):
                vals = plsc.load_gather(inbufs[s], [rowpats[fg], cols])
                tbufs[s][q, pl.ds(h + fg * 16, 16)] = vals

    def write_chunk(s, t):
        pltpu.async_copy(
            tbufs[s], out_hbm.at[pl.ds(t * (CC // 2), CC // 2)], osems[s])

    def wait_write(s):
        pltpu.make_async_copy(
            tbufs[s], out_hbm.at[pl.ds(0, CC // 2)], osems[s]).wait()

    issue_in(0, w)

    @pl.when(w + NW < NCHK)
    def _():
        issue_in(1, w + NW)

    def lap(i, carry):
        for s in range(2):
            t = (2 * i + s) * NW + w

            @pl.when(t < NCHK)
            def _():
                @pl.when(2 * i + s >= 2)
                def _():
                    wait_write(s)
                wait_in(s)
                transpose_chunk(s)
                write_chunk(s, t)
                nt = t + 2 * NW

                @pl.when(nt < NCHK)
                def _():
                    issue_in(s, nt)
        return carry

    lax.fori_loop(0, (ITER + 1) // 2, lap, 0)

    for s in range(2):
        @pl.when(w + s * NW < NCHK)
        def _():
            wait_write(s)

    @pl.when(w == 0)
    def _():
        for r in range(8):
            pltpu.async_copy(
                tail_hbm.at[pl.ds(8 * r, 8), pl.ds(0, 128)],
                tailb.at[pl.ds(8 * r, 8), pl.ds(0, 128)], i0)
        for r in range(8):
            pltpu.make_async_copy(
                tail_hbm.at[pl.ds(0, 8), pl.ds(0, 128)],
                tailb.at[pl.ds(0, 8), pl.ds(0, 128)], i0).wait()

        @plsc.parallel_loop(0, TAIL, unroll=2)
        def tail_body(c):
            q = c >> 1
            h = (c & 1) * 64
            cols = zeros16 + c
            for fg in range(D // 16):
                vals = plsc.load_gather(tailb, [rowpats[fg], cols])
                tb0[q, pl.ds(h + fg * 16, 16)] = vals

        pltpu.async_copy(
            tb0.at[pl.ds(0, TAIL // 2)],
            out_hbm.at[pl.ds(MAIN // 2, TAIL // 2)], o0)
        pltpu.make_async_copy(
            tb0.at[pl.ds(0, TAIL // 2)],
            out_hbm.at[pl.ds(0, TAIL // 2)], o0).wait()


_g_scratch = (
    [pltpu.VMEM((UPW, CHUNK), jnp.int32)]
    + [pltpu.VMEM((UPW, CHUNK), jnp.int32)]
    + [pltpu.VMEM((CHUNK, 128), jnp.float32)] * 2
    + [pltpu.VMEM((16, CHUNK + 1), jnp.float32)] * (2 * NFG)
    + [pltpu.SemaphoreType.DMA] * 2
    + [pltpu.SemaphoreType.DMA] * 2
)


@functools.partial(
    pl.kernel,
    mesh=_mesh,
    out_type=jax.ShapeDtypeStruct((OUT_ELEMS // CHUNK, CHUNK), jnp.float32),
    scratch_types=_g_scratch,
    compiler_params=pltpu.CompilerParams(needs_layout_passes=False),
)
def _gather_k(pairs_hbm, x_hbm, par_hbm, out_hbm, idx_v, par_v, *rest):
    gbufs = rest[0:2]
    tbufs = (rest[2:2 + NFG], rest[2 + NFG:2 + 2 * NFG])
    gsems = rest[2 + 2 * NFG:4 + 2 * NFG]
    wsems = rest[4 + 2 * NFG:6 + 2 * NFG]

    w = lax.axis_index("s") * NC + lax.axis_index("c")
    ubase = w * UPW
    pltpu.sync_copy(x_hbm.at[pl.ds(ubase, UPW)], idx_v)
    pltpu.sync_copy(par_hbm.at[pl.ds(ubase, UPW)], par_v)

    rows16 = lax.iota(jnp.int32, 16)
    zeros16 = rows16 * 0

    def wait_gather(s):
        pltpu.make_async_copy(
            pairs_hbm.at[idx_v.at[0]], gbufs[s], gsems[s]).wait()

    def wait_writes(s):
        for fg in range(NFG):
            for _ in range(2):
                pltpu.make_async_copy(
                    tbufs[s][fg].at[pl.ds(0, 8), pl.ds(0, CHUNK)],
                    out_hbm.at[pl.ds(0, 8)], wsems[s]).wait()

    def transpose_unit(s, u):
        usplat = zeros16 + u

        @plsc.parallel_loop(0, CHUNK, unroll=8)
        def b_body(b):
            cols = zeros16 + b
            pbase = plsc.load_gather(par_v, [usplat, cols]) + rows16
            for fg in range(NFG):
                vals = plsc.load_gather(gbufs[s], [cols, pbase + fg * 16])
                plsc.store_scatter(tbufs[s][fg], [rows16, cols], vals)

    def write_unit(s, u):
        j = u // (BATCH // CHUNK)
        c = u % (BATCH // CHUNK)
        rbase = j * (64 * 128) + c * 8
        for r in range(8):
            pltpu.async_copy(
                tbufs[s][r // 2].at[pl.ds(8 * r % 16, 8), pl.ds(0, CHUNK)],
                out_hbm.at[pl.ds(rbase + r * (128 * 8), 8)],
                wsems[s])

    def issue_gather(s, u):
        pltpu.async_copy(pairs_hbm.at[idx_v.at[u]], gbufs[s], gsems[s])

    issue_gather(0, 0)
    issue_gather(1, 1)
    for s in range(2):
        wait_gather(s)
        transpose_unit(s, s)
        write_unit(s, ubase + s)
        issue_gather(s, s + 2)

    def lap(L, carry):
        for s in range(2):
            u = 2 * L + s
            wait_writes(s)
            wait_gather(s)
            transpose_unit(s, u)
            write_unit(s, ubase + u)
            issue_gather(s, u + 2)
        return carry

    lax.fori_loop(1, UPW // 2 - 1, lap, 0)

    for s in range(2):
        u = UPW - 2 + s
        wait_writes(s)
        wait_gather(s)
        transpose_unit(s, u)
        write_unit(s, ubase + u)

    for s in range(2):
        wait_writes(s)


def kernel(x, table):
    tail = jnp.pad(table[MAIN:, :].T, ((0, 0), (0, 128 - TAIL)))
    pairs = _repack_k(table.T, tail)
    xi = x.astype(jnp.int32)
    idx = (xi >> 1).T.reshape(NUNITS, CHUNK)
    par = ((xi & 1) * 64).T.reshape(NUNITS, CHUNK)
    out = _gather_k(pairs, idx, par)
    o = out.reshape(FIELDS, 8, 128, 8, 128)
    o = o.transpose(2, 4, 0, 1, 3)
    return o.reshape(BATCH, FIELDS, D)


# speedup vs baseline: 1.2303x; 1.0034x over previous
import functools

import jax
import jax.numpy as jnp
from jax import lax
from jax.experimental import pallas as pl
from jax.experimental.pallas import tpu as pltpu
from jax.experimental.pallas import tpu_sc as plsc

BATCH = 16384
FIELDS = 26
D = 64
ACT = 1000000
NC, NS = 2, 16
NW = NC * NS
OUT_ELEMS = BATCH * FIELDS * D

CC = 256
MAIN = 999936
NCHK = MAIN // CC
TAIL = ACT - MAIN
NPAIR = ACT // 2
ITER = (NCHK + NW - 1) // NW + 1

CHUNK = 128
NUNITS = FIELDS * (BATCH // CHUNK)
UPW = NUNITS // NW
NFG = D // 16

_mesh = plsc.VectorSubcoreMesh(core_axis_name="c", subcore_axis_name="s")

_r_scratch = (
    [pltpu.VMEM((D, CC + 1), jnp.float32)] * 2
    + [pltpu.VMEM((CC // 2, 128), jnp.float32)] * 2
    + [pltpu.VMEM((D, 129), jnp.float32)]
    + [pltpu.SemaphoreType.DMA] * 2
    + [pltpu.SemaphoreType.DMA] * 2
)


@functools.partial(
    pl.kernel,
    mesh=_mesh,
    out_type=jax.ShapeDtypeStruct((NPAIR, 128), jnp.float32),
    scratch_types=_r_scratch,
    compiler_params=pltpu.CompilerParams(needs_layout_passes=False),
)
def _repack_k(tT_hbm, tail_hbm, out_hbm, in0, in1, tb0, tb1, tailb,
              i0, i1, o0, o1):
    w = lax.axis_index("s") * NC + lax.axis_index("c")
    inbufs = (in0, in1)
    tbufs = (tb0, tb1)
    isems = (i0, i1)
    osems = (o0, o1)

    it = lax.iota(jnp.int32, 16)
    zeros16 = it * 0
    rowpats = [it + 16 * fg for fg in range(D // 16)]

    def issue_in(s, t):
        pltpu.async_copy(
            tT_hbm.at[pl.ds(0, D), pl.ds(t * CC, CC)],
            inbufs[s].at[pl.ds(0, D), pl.ds(0, CC)], isems[s])

    def wait_in(s):
        pltpu.make_async_copy(
            tT_hbm.at[pl.ds(0, D), pl.ds(0, CC)],
            inbufs[s].at[pl.ds(0, D), pl.ds(0, CC)], isems[s]).wait()

    def transpose_chunk(s):
        @plsc.parallel_loop(0, CC, unroll=4)
        def c_body(c):
            q = c >> 1
            h = (c & 1) * 64
            cols = zeros16 + c
            for fg in range(D // 16):
                vals = plsc.load_gather(inbufs[s], [rowpats[fg], cols])
                tbufs[s][q, pl.ds(h + fg * 16, 16)] = vals

    def write_chunk(s, t):
        pltpu.async_copy(
            tbufs[s], out_hbm.at[pl.ds(t * (CC // 2), CC // 2)], osems[s])

    def wait_write(s):
        pltpu.make_async_copy(
            tbufs[s], out_hbm.at[pl.ds(0, CC // 2)], osems[s]).wait()

    issue_in(0, w)

    @pl.when(w + NW < NCHK)
    def _():
        issue_in(1, w + NW)

    def lap(i, carry):
        for s in range(2):
            t = (2 * i + s) * NW + w

            @pl.when(t < NCHK)
            def _():
                @pl.when(2 * i + s >= 2)
                def _():
                    wait_write(s)
                wait_in(s)
                transpose_chunk(s)
                write_chunk(s, t)
                nt = t + 2 * NW

                @pl.when(nt < NCHK)
                def _():
                    issue_in(s, nt)
        return carry

    lax.fori_loop(0, (ITER + 1) // 2, lap, 0)

    for s in range(2):
        @pl.when(w + s * NW < NCHK)
        def _():
            wait_write(s)

    @pl.when(w == 0)
    def _():
        for r in range(8):
            pltpu.async_copy(
                tail_hbm.at[pl.ds(8 * r, 8), pl.ds(0, 128)],
                tailb.at[pl.ds(8 * r, 8), pl.ds(0, 128)], i0)
        for r in range(8):
            pltpu.make_async_copy(
                tail_hbm.at[pl.ds(0, 8), pl.ds(0, 128)],
                tailb.at[pl.ds(0, 8), pl.ds(0, 128)], i0).wait()

        @plsc.parallel_loop(0, TAIL, unroll=2)
        def tail_body(c):
            q = c >> 1
            h = (c & 1) * 64
            cols = zeros16 + c
            for fg in range(D // 16):
                vals = plsc.load_gather(tailb, [rowpats[fg], cols])
                tb0[q, pl.ds(h + fg * 16, 16)] = vals

        pltpu.async_copy(
            tb0.at[pl.ds(0, TAIL // 2)],
            out_hbm.at[pl.ds(MAIN // 2, TAIL // 2)], o0)
        pltpu.make_async_copy(
            tb0.at[pl.ds(0, TAIL // 2)],
            out_hbm.at[pl.ds(0, TAIL // 2)], o0).wait()


_g_scratch = (
    [pltpu.VMEM((UPW, CHUNK), jnp.int32)]
    + [pltpu.VMEM((16 * CHUNK,), jnp.int32)] * 2
    + [pltpu.VMEM((CHUNK, 128), jnp.float32)] * 2
    + [pltpu.VMEM((16, CHUNK + 1), jnp.float32)] * (2 * NFG)
    + [pltpu.SemaphoreType.DMA] * 2
    + [pltpu.SemaphoreType.DMA] * 2
)


@functools.partial(
    pl.kernel,
    mesh=_mesh,
    out_type=jax.ShapeDtypeStruct((OUT_ELEMS // CHUNK, CHUNK), jnp.float32),
    scratch_types=_g_scratch,
    compiler_params=pltpu.CompilerParams(needs_layout_passes=False),
)
def _gather_k(pairs_hbm, x_hbm, par_hbm, out_hbm, idx_v, *rest):
    parbufs = rest[0:2]
    rest = rest[2:]
    gbufs = rest[0:2]
    tbufs = (rest[2:2 + NFG], rest[2 + NFG:2 + 2 * NFG])
    gsems = rest[2 + 2 * NFG:4 + 2 * NFG]
    wsems = rest[4 + 2 * NFG:6 + 2 * NFG]

    w = lax.axis_index("s") * NC + lax.axis_index("c")
    ubase = w * UPW
    pltpu.sync_copy(x_hbm.at[pl.ds(ubase, UPW)], idx_v)

    rows16 = lax.iota(jnp.int32, 16)
    zeros16 = rows16 * 0

    def wait_gather(s):
        pltpu.make_async_copy(
            pairs_hbm.at[idx_v.at[0]], gbufs[s], gsems[s]).wait()
        pltpu.make_async_copy(
            par_hbm.at[pl.ds(0, 16 * CHUNK)], parbufs[s], gsems[s]).wait()

    def wait_writes(s):
        for fg in range(NFG):
            for _ in range(2):
                pltpu.make_async_copy(
                    tbufs[s][fg].at[pl.ds(0, 8), pl.ds(0, CHUNK)],
                    out_hbm.at[pl.ds(0, 8)], wsems[s]).wait()

    def transpose_unit(s):
        @plsc.parallel_loop(0, CHUNK, unroll=8)
        def b_body(b):
            cols = zeros16 + b
            pbase = parbufs[s][pl.ds(16 * b, 16)] + rows16
            for fg in range(NFG):
                vals = plsc.load_gather(gbufs[s], [cols, pbase + fg * 16])
                plsc.store_scatter(tbufs[s][fg], [rows16, cols], vals)

    def write_unit(s, u):
        j = u // (BATCH // CHUNK)
        c = u % (BATCH // CHUNK)
        rbase = j * (64 * 128) + c * 8
        for r in range(8):
            pltpu.async_copy(
                tbufs[s][r // 2].at[pl.ds(8 * r % 16, 8), pl.ds(0, CHUNK)],
                out_hbm.at[pl.ds(rbase + r * (128 * 8), 8)],
                wsems[s])

    def issue_gather(s, u):
        pltpu.async_copy(pairs_hbm.at[idx_v.at[u]], gbufs[s], gsems[s])
        pltpu.async_copy(
            par_hbm.at[pl.ds((ubase + u) * 16 * CHUNK, 16 * CHUNK)],
            parbufs[s], gsems[s])

    issue_gather(0, 0)
    issue_gather(1, 1)
    for s in range(2):
        wait_gather(s)
        transpose_unit(s)
        write_unit(s, ubase + s)
        issue_gather(s, s + 2)

    def lap(L, carry):
        for s in range(2):
            u = 2 * L + s
            wait_writes(s)
            wait_gather(s)
            transpose_unit(s)
            write_unit(s, ubase + u)
            issue_gather(s, u + 2)
        return carry

    lax.fori_loop(1, UPW // 2 - 1, lap, 0)

    for s in range(2):
        u = UPW - 2 + s
        wait_writes(s)
        wait_gather(s)
        transpose_unit(s)
        write_unit(s, ubase + u)

    for s in range(2):
        wait_writes(s)


def kernel(x, table):
    tail = jnp.pad(table[MAIN:, :].T, ((0, 0), (0, 128 - TAIL)))
    pairs = _repack_k(table.T, tail)
    xi = x.astype(jnp.int32)
    idx = (xi >> 1).T.reshape(NUNITS, CHUNK)
    par = jnp.broadcast_to(
        ((xi & 1) * 64).T.reshape(NUNITS, CHUNK, 1),
        (NUNITS, CHUNK, 16)).reshape(NUNITS * CHUNK * 16)
    out = _gather_k(pairs, idx, par)
    o = out.reshape(FIELDS, 8, 128, 8, 128)
    o = o.transpose(2, 4, 0, 1, 3)
    return o.reshape(BATCH, FIELDS, D)


# speedup vs baseline: 2.1100x; 1.7151x over previous
import functools

import jax
import jax.numpy as jnp
from jax import lax
from jax.experimental import pallas as pl
from jax.experimental.pallas import tpu as pltpu
from jax.experimental.pallas import tpu_sc as plsc

BATCH = 16384
FIELDS = 26
D = 64
NC, NS = 2, 16
NW = NC * NS
CHUNK = 128
NUNITS = FIELDS * (BATCH // CHUNK)
UPW = NUNITS // NW
NFG = D // 16
OUT_ELEMS = BATCH * FIELDS * D

_mesh = plsc.VectorSubcoreMesh(core_axis_name="c", subcore_axis_name="s")

_scratch = (
    [pltpu.VMEM((UPW, CHUNK), jnp.int32)]
    + [pltpu.VMEM((CHUNK, D), jnp.float32)] * 2
    + [pltpu.VMEM((16, CHUNK + 1), jnp.float32)] * (2 * NFG)
    + [pltpu.SemaphoreType.DMA] * 2
    + [pltpu.SemaphoreType.DMA] * 2
)


@functools.partial(
    pl.kernel,
    mesh=_mesh,
    out_type=jax.ShapeDtypeStruct((OUT_ELEMS // CHUNK, CHUNK), jnp.float32),
    scratch_types=_scratch,
    compiler_params=pltpu.CompilerParams(
        needs_layout_passes=False, use_tc_tiling_on_sc=False),
)
def _gather_k(table_hbm, x_hbm, out_hbm, idx_v, *rest):
    gbufs = rest[0:2]
    tbufs = (rest[2:2 + NFG], rest[2 + NFG:2 + 2 * NFG])
    gsems = rest[2 + 2 * NFG:4 + 2 * NFG]
    wsems = rest[4 + 2 * NFG:6 + 2 * NFG]

    w = lax.axis_index("s") * NC + lax.axis_index("c")
    ubase = w * UPW
    pltpu.sync_copy(x_hbm.at[pl.ds(ubase, UPW)], idx_v)

    rows16 = lax.iota(jnp.int32, 16)
    zeros16 = rows16 * 0

    def wait_gather(s):
        pltpu.make_async_copy(
            table_hbm.at[idx_v.at[0]], gbufs[s], gsems[s]).wait()

    def wait_writes(s):
        for fg in range(NFG):
            for _ in range(2):
                pltpu.make_async_copy(
                    tbufs[s][fg].at[pl.ds(0, 8), pl.ds(0, CHUNK)],
                    out_hbm.at[pl.ds(0, 8)], wsems[s]).wait()

    def transpose_unit(s):
        @plsc.parallel_loop(0, CHUNK, unroll=8)
        def b_body(b):
            cols = zeros16 + b
            for fg in range(NFG):
                vals = gbufs[s][b, pl.ds(fg * 16, 16)]
                plsc.store_scatter(tbufs[s][fg], [rows16, cols], vals)

    def write_unit(s, u):
        j = u // (BATCH // CHUNK)
        c = u % (BATCH // CHUNK)
        rbase = j * (64 * 128) + c * 8
        for r in range(8):
            pltpu.async_copy(
                tbufs[s][r // 2].at[pl.ds(8 * r % 16, 8), pl.ds(0, CHUNK)],
                out_hbm.at[pl.ds(rbase + r * (128 * 8), 8)],
                wsems[s])

    def issue_gather(s, u):
        pltpu.async_copy(table_hbm.at[idx_v.at[u]], gbufs[s], gsems[s])

    issue_gather(0, 0)
    issue_gather(1, 1)
    for s in range(2):
        wait_gather(s)
        transpose_unit(s)
        write_unit(s, ubase + s)
        issue_gather(s, s + 2)

    def lap(L, carry):
        for s in range(2):
            u = 2 * L + s
            wait_writes(s)
            wait_gather(s)
            transpose_unit(s)
            write_unit(s, ubase + u)
            issue_gather(s, u + 2)
        return carry

    lax.fori_loop(1, UPW // 2 - 1, lap, 0)

    for s in range(2):
        u = UPW - 2 + s
        wait_writes(s)
        wait_gather(s)
        transpose_unit(s)
        write_unit(s, ubase + u)

    for s in range(2):
        wait_writes(s)


def kernel(x, table):
    idx = x.astype(jnp.int32).T.reshape(NUNITS, CHUNK)
    out = _gather_k(table, idx)
    o = out.reshape(FIELDS, 8, 128, 8, 128)
    o = o.transpose(2, 4, 0, 1, 3)
    return o.reshape(BATCH, FIELDS, D)
